# Initial kernel scaffold; baseline (speedup 1.0000x reference)
import functools

import jax
import jax.numpy as jnp
from jax.experimental import pallas as pl
from jax.experimental.pallas import tpu as pltpu

N = 10000
E = 320000
K = 16
D_X = 128
D_EN = 16


def _mlp(h, params):
    for W, b in params[:-1]:
        h = jax.nn.relu(h @ W.T + b)
    W, b = params[-1]
    return h @ W.T + b


def _prep_kernel(x_ref, en_ref, tw_ref, tb_ref, pw_ref, pb_ref,
                 w0, b0, w1, b1, w2, b2, w3, b3,
                 a_ref, bsrc_ref, p_ref):
    x = x_ref[...]
    tw = tw_ref[...]
    pw = pw_ref[...]
    a_ref[...] = jnp.dot(x, tw.T, preferred_element_type=jnp.float32) \
        + (tb_ref[...] + pb_ref[...])[None, :]
    bsrc_ref[...] = jnp.dot(x, (pw - tw).T, preferred_element_type=jnp.float32)
    en = en_ref[...]
    h = en
    for (w, b) in ((w0, b0), (w1, b1), (w2, b2)):
        h = jax.nn.relu(jnp.dot(h, w[...].T, preferred_element_type=jnp.float32) + b[...][None, :])
    p_ref[...] = jnp.dot(h, w3[...].T, preferred_element_type=jnp.float32) + b3[...][None, :]


def _prep(x, en, theta_W, theta_b, phi_W, phi_b, phi_en_params):
    flat = []
    for w, b in phi_en_params:
        flat += [w, b]
    return pl.pallas_call(
        _prep_kernel,
        out_shape=(
            jax.ShapeDtypeStruct((N, D_X), jnp.float32),
            jax.ShapeDtypeStruct((N, D_X), jnp.float32),
            jax.ShapeDtypeStruct((N, D_EN), jnp.float32),
        ),
    )(x, en, theta_W, theta_b, phi_W, phi_b, *flat)


def kernel(x, en, theta_W, theta_b, phi_W, phi_b, theta_en_params, phi_en_params, edge_index):
    src = edge_index[0]
    dst = edge_index[1]
    A, B, p = _prep(x, en, theta_W, theta_b, phi_W, phi_b, phi_en_params)

    dif = jnp.take(en, dst, axis=0) - jnp.take(en, src, axis=0)
    h = _mlp(dif, theta_en_params) + jnp.take(p, src, axis=0)
    deg = jax.ops.segment_sum(jnp.ones((E,), jnp.float32), dst, num_segments=N)
    S = jax.ops.segment_sum(h, dst, num_segments=N)
    en_new = S / jnp.maximum(deg, 1.0)[:, None]

    M = jax.ops.segment_max(jnp.take(B, src, axis=0), dst, num_segments=N)
    x_new = jnp.where((deg > 0.0)[:, None], A + M, 0.0)

    sq = jnp.sum(x_new * x_new, axis=1)
    dist = sq[:, None] + sq[None, :] - 2.0 * (x_new @ x_new.T)
    _, nbr = jax.lax.top_k(-dist, K)
    src_new = nbr.reshape(-1)
    dst_new = jnp.repeat(jnp.arange(N, dtype=src_new.dtype), K)
    edge_index_new = jnp.stack([src_new, dst_new])
    return x_new, en_new, edge_index_new


# speedup vs baseline: 1.0629x; 1.0629x over previous
import functools

import jax
import jax.numpy as jnp
from jax.experimental import pallas as pl
from jax.experimental.pallas import tpu as pltpu

N = 10000
E = 320000
K = 16
D_X = 128
D_EN = 16


def _mlp(h, params):
    for W, b in params[:-1]:
        h = jax.nn.relu(h @ W.T + b)
    W, b = params[-1]
    return h @ W.T + b


def _prep_kernel(x_ref, en_ref, tw_ref, tb_ref, pw_ref, pb_ref,
                 w0, b0, w1, b1, w2, b2, w3, b3,
                 a_ref, bsrc_ref, p_ref):
    x = x_ref[...]
    tw = tw_ref[...]
    pw = pw_ref[...]
    hi = jax.lax.Precision.HIGHEST
    a_ref[...] = jnp.dot(x, tw.T, preferred_element_type=jnp.float32, precision=hi) \
        + (tb_ref[...] + pb_ref[...])[None, :]
    bsrc_ref[...] = jnp.dot(x, (pw - tw).T, preferred_element_type=jnp.float32, precision=hi)
    en = en_ref[...]
    h = en
    for (w, b) in ((w0, b0), (w1, b1), (w2, b2)):
        h = jax.nn.relu(jnp.dot(h, w[...].T, preferred_element_type=jnp.float32, precision=hi) + b[...][None, :])
    p_ref[...] = jnp.dot(h, w3[...].T, preferred_element_type=jnp.float32, precision=hi) + b3[...][None, :]


def _prep(x, en, theta_W, theta_b, phi_W, phi_b, phi_en_params):
    flat = []
    for w, b in phi_en_params:
        flat += [w, b]
    return pl.pallas_call(
        _prep_kernel,
        out_shape=(
            jax.ShapeDtypeStruct((N, D_X), jnp.float32),
            jax.ShapeDtypeStruct((N, D_X), jnp.float32),
            jax.ShapeDtypeStruct((N, D_EN), jnp.float32),
        ),
    )(x, en, theta_W, theta_b, phi_W, phi_b, *flat)


def kernel(x, en, theta_W, theta_b, phi_W, phi_b, theta_en_params, phi_en_params, edge_index):
    src = edge_index[0]
    dst = edge_index[1]
    A, B, p = _prep(x, en, theta_W, theta_b, phi_W, phi_b, phi_en_params)

    dif = jnp.take(en, dst, axis=0) - jnp.take(en, src, axis=0)
    h = _mlp(dif, theta_en_params) + jnp.take(p, src, axis=0)
    deg = jax.ops.segment_sum(jnp.ones((E,), jnp.float32), dst, num_segments=N)
    S = jax.ops.segment_sum(h, dst, num_segments=N)
    en_new = S / jnp.maximum(deg, 1.0)[:, None]

    M = jax.ops.segment_max(jnp.take(B, src, axis=0), dst, num_segments=N)
    x_new = jnp.where((deg > 0.0)[:, None], A + M, 0.0)

    sq = jnp.sum(x_new * x_new, axis=1)
    dist = sq[:, None] + sq[None, :] - 2.0 * (x_new @ x_new.T)
    _, nbr = jax.lax.top_k(-dist, K)
    src_new = nbr.reshape(-1)
    dst_new = jnp.repeat(jnp.arange(N, dtype=src_new.dtype), K)
    edge_index_new = jnp.stack([src_new, dst_new])
    return x_new, en_new, edge_index_new


# speedup vs baseline: 1.2849x; 1.2088x over previous
import functools

import jax
import jax.numpy as jnp
from jax import lax
from jax.experimental import pallas as pl
from jax.experimental.pallas import tpu as pltpu, tpu_sc as plsc

N = 10000
E = 320000
K = 16
DX = 128
DEN = 16

NC = 2
NS = 16
NW = NC * NS
EW = E // NW

NPAD = 10240
NCHUNK = 80
RPT = 320
TAIL = N - 31 * RPT
BIGNEG = -3.0e38
BIGPOS = 3.0e38

def _mesh():
    return plsc.VectorSubcoreMesh(core_axis_name="c", subcore_axis_name="s")


def _wid():
    return lax.axis_index("s") * NC + lax.axis_index("c")


def _prep_body(x_ref, en_ref, pw_ref, pb_ref, w0, b0, w1, b1, w2, b2, w3, b3,
               p2_ref, cdst_ref, csrc_ref):
    x = x_ref[...]
    p2_ref[...] = jnp.dot(x, pw_ref[...].T, preferred_element_type=jnp.float32) + pb_ref[...]
    en = en_ref[...]
    h = en
    for (w, b) in ((w0, b0), (w1, b1), (w2, b2)):
        h = jax.nn.relu(jnp.dot(h, w[...].T, preferred_element_type=jnp.float32) + b[...])
    p = jnp.dot(h, w3[...].T, preferred_element_type=jnp.float32) + b3[...]
    z96 = jnp.zeros((N, 96), jnp.float32)
    z112 = jnp.zeros((N, 112), jnp.float32)
    cdst_ref[...] = jnp.concatenate([en, z112], axis=1)
    csrc_ref[...] = jnp.concatenate([-en, p, z96], axis=1)


def _prep(x, en, phi_W, phi_b2, pflat):
    return pl.pallas_call(
        _prep_body,
        out_shape=(
            jax.ShapeDtypeStruct((N, DX), jnp.float32),
            jax.ShapeDtypeStruct((N, DX), jnp.float32),
            jax.ShapeDtypeStruct((N, DX), jnp.float32),
        ),
    )(x, en, phi_W, phi_b2, *pflat)


GCH = 400


@functools.lru_cache(maxsize=1)
def _mk_sc_gather():
  return functools.partial(
    pl.kernel,
    out_type=[jax.ShapeDtypeStruct((E, DX), jnp.float32),
              jax.ShapeDtypeStruct((E, DX), jnp.float32),
              jax.ShapeDtypeStruct((E, DX), jnp.float32)],
    mesh=_mesh(),
    scratch_types=[pltpu.VMEM((GCH,), jnp.int32),
                   pltpu.VMEM((GCH,), jnp.int32),
                   pltpu.VMEM((GCH, DX), jnp.float32),
                   pltpu.SemaphoreType.DMA],
  )(_sc_gather)


def _sc_gather(x_hbm, xneg_hbm, cdst_hbm, csrc_hbm, p2_hbm, src_hbm, dst_hbm,
               xd_hbm, f_hbm, p2s_hbm, sidx, didx, rows, sem):
    base0 = _wid() * EW

    def chunk(c, carry):
        base = base0 + c * GCH
        pltpu.sync_copy(src_hbm.at[pl.ds(base, GCH)], sidx)
        pltpu.sync_copy(dst_hbm.at[pl.ds(base, GCH)], didx)
        pltpu.async_copy(cdst_hbm.at[didx], rows, sem).wait()
        pltpu.async_copy(csrc_hbm.at[sidx], rows, sem, add=True).wait()
        pltpu.sync_copy(rows, f_hbm.at[pl.ds(base, GCH)])
        pltpu.async_copy(x_hbm.at[didx], rows, sem).wait()
        pltpu.async_copy(xneg_hbm.at[sidx], rows, sem, add=True).wait()
        pltpu.sync_copy(rows, xd_hbm.at[pl.ds(base, GCH)])
        pltpu.async_copy(p2_hbm.at[sidx], rows, sem).wait()
        pltpu.sync_copy(rows, p2s_hbm.at[pl.ds(base, GCH)])
        return carry

    lax.fori_loop(0, EW // GCH, chunk, jnp.int32(0))


BE = 8000


def _edge_body(xd_ref, f_ref, p2s_ref, tw_ref, tb_ref, w0, b0, w1, b1, w2, b2, w3, b3,
               exx_ref, hh_ref):
    exx_ref[...] = (jnp.dot(xd_ref[...], tw_ref[...].T,
                            preferred_element_type=jnp.float32) + tb_ref[...]) + p2s_ref[...]
    f = f_ref[...]
    h = f[:, 0:DEN]
    for (w, b) in ((w0, b0), (w1, b1), (w2, b2)):
        h = jax.nn.relu(jnp.dot(h, w[...].T, preferred_element_type=jnp.float32) + b[...])
    h = jnp.dot(h, w3[...].T, preferred_element_type=jnp.float32) + b3[...]
    h = h + f[:, DEN:2 * DEN]
    ones = jnp.ones((BE, 1), jnp.float32)
    zeros = jnp.zeros((BE, DX - DEN - 1), jnp.float32)
    hh_ref[...] = jnp.concatenate([h, ones, zeros], axis=1)


def _edge(xd, F, p2s, theta_W, theta_b2, tflat):
    nb = E // BE
    bs_e = pl.BlockSpec((BE, DX), lambda i: (i, 0))
    specs = [bs_e, bs_e, bs_e, pl.BlockSpec(theta_W.shape, lambda i: (0, 0)),
             pl.BlockSpec(theta_b2.shape, lambda i: (0, 0))]
    for t in tflat:
        specs.append(pl.BlockSpec(t.shape, lambda i: (0, 0)))
    return pl.pallas_call(
        _edge_body,
        grid=(nb,),
        in_specs=specs,
        out_specs=(bs_e, bs_e),
        out_shape=(jax.ShapeDtypeStruct((E, DX), jnp.float32),
                   jax.ShapeDtypeStruct((E, DX), jnp.float32)),
    )(xd, F, p2s, theta_W, theta_b2, *tflat)


SCH = 400
HALF = 5000
AROWS = 5008
ZR2 = 312


@functools.lru_cache(maxsize=1)
def _mk_sc_scatter():
  return functools.partial(
    pl.kernel,
    out_type=jax.ShapeDtypeStruct((NC, N, DX), jnp.float32),
    mesh=_mesh(),
    scratch_types=[pltpu.VMEM((SCH,), jnp.int32),
                   pltpu.VMEM((SCH,), jnp.int32),
                   pltpu.VMEM((SCH, DX), jnp.float32),
                   pltpu.VMEM_SHARED((AROWS, DX), jnp.float32),
                   pltpu.SemaphoreType.DMA],
  )(_sc_scatter)


def _sc_scatter(hh_hbm, dst_hbm, z_hbm, out_hbm, didx, lidx, rows, acc, sem):
    cid = lax.axis_index("c")
    sid = lax.axis_index("s")
    zbase = sid * ZR2
    base0 = _wid() * EW

    for p in range(2):
        pltpu.sync_copy(z_hbm.at[pl.ds(zbase, ZR2)], acc.at[pl.ds(zbase, ZR2)])

        @pl.when(sid == 0)
        def _():
            pltpu.sync_copy(z_hbm.at[pl.ds(NS * ZR2, AROWS - NS * ZR2)],
                            acc.at[pl.ds(NS * ZR2, AROWS - NS * ZR2)])

        plsc.subcore_barrier()
        lov = jnp.full((16,), HALF * p, jnp.int32)
        hiv = lov + HALF
        trash = jnp.full((16,), HALF, jnp.int32)

        def chunk(c, carry):
            base = base0 + c * SCH
            pltpu.sync_copy(dst_hbm.at[pl.ds(base, SCH)], didx)
            pltpu.sync_copy(hh_hbm.at[pl.ds(base, SCH)], rows)

            def remap(j, c2):
                d16 = didx[pl.ds(j * 16, 16)]
                ok = (d16 >= lov) & (d16 < hiv)
                lidx[pl.ds(j * 16, 16)] = jnp.where(ok, d16 - lov, trash)
                return c2

            lax.fori_loop(0, SCH // 16, remap, jnp.int32(0))
            pltpu.async_copy(rows, acc.at[lidx], sem, add=True).wait()
            return carry

        lax.fori_loop(0, EW // SCH, chunk, jnp.int32(0))
        plsc.subcore_barrier()
        obase = p * HALF + zbase
        pltpu.sync_copy(acc.at[pl.ds(zbase, ZR2)], out_hbm.at[cid, pl.ds(obase, ZR2)])

        @pl.when(sid == 0)
        def _():
            pltpu.sync_copy(acc.at[pl.ds(NS * ZR2, HALF - NS * ZR2)],
                            out_hbm.at[cid, pl.ds(p * HALF + NS * ZR2, HALF - NS * ZR2)])

        plsc.subcore_barrier()


def _finx_body(m_ref, xnew_ref, xpad_ref):
    m = m_ref[...]
    xn = jnp.where(m <= BIGNEG / 2, 0.0, m)
    xnew_ref[...] = xn
    xpad_ref[0:N, :] = xn
    xpad_ref[N:NPAD, :] = jnp.zeros((NPAD - N, DX), jnp.float32)


def _finx(M):
    return pl.pallas_call(
        _finx_body,
        out_shape=(jax.ShapeDtypeStruct((N, DX), jnp.float32),
                   jax.ShapeDtypeStruct((NPAD, DX), jnp.float32)),
    )(M)


def _finen_body(sd_ref, en_ref):
    s = sd_ref[0] + sd_ref[1]
    deg = s[:, DEN:DEN + 1]
    en_ref[...] = s[:, 0:DEN] / jnp.maximum(deg, 1.0)


def _finen(SD):
    return pl.pallas_call(
        _finen_body,
        out_shape=jax.ShapeDtypeStruct((N, DEN), jnp.float32),
    )(SD)


RB = 200


def _dist_body(xb_ref, xc_ref, d_ref, cm_ref):
    xb = xb_ref[...]
    sqb = jnp.sum(xb * xb, axis=1, keepdims=True)
    lane = lax.broadcasted_iota(jnp.int32, (RB, 128), 1)
    for c in range(NCHUNK):
        xc = xc_ref[pl.ds(c * 128, 128), :]
        g = jnp.dot(xb, xc.T, preferred_element_type=jnp.float32)
        sqc = jnp.transpose(jnp.sum(xc * xc, axis=1, keepdims=True))
        d = (sqb + sqc) - 2.0 * g
        if c == NCHUNK - 2:
            d = jnp.where(lane < 16, d, BIGPOS)
        elif c == NCHUNK - 1:
            d = jnp.full((RB, 128), BIGPOS, jnp.float32)
        d_ref[:, c, :] = d
        cm_ref[:, c:c + 1] = jnp.min(d, axis=1, keepdims=True)
    cm_ref[:, NCHUNK:128] = jnp.full((RB, 128 - NCHUNK), BIGPOS, jnp.float32)


def _dist(xpad):
    nb = N // RB
    return pl.pallas_call(
        _dist_body,
        grid=(nb,),
        in_specs=[pl.BlockSpec((RB, DX), lambda i: (i, 0)),
                  pl.BlockSpec((NPAD, DX), lambda i: (0, 0))],
        out_specs=(pl.BlockSpec((RB, NCHUNK, 128), lambda i: (i, 0, 0)),
                   pl.BlockSpec((RB, 128), lambda i: (i, 0))),
        out_shape=(jax.ShapeDtypeStruct((N, NCHUNK, 128), jnp.float32),
                   jax.ShapeDtypeStruct((N, 128), jnp.float32)),
    )(xpad, xpad)


def _cmsel_body(cm_ref, id_ref):
    cm = cm_ref[...]
    lane = lax.broadcasted_iota(jnp.int32, (RB, 128), 1)
    mask = jnp.ones((RB, 128), jnp.bool_)
    ids = jnp.zeros((RB, 128), jnp.int32)
    for k in range(K):
        cur = jnp.where(mask, cm, BIGPOS)
        mv = jnp.min(cur, axis=1, keepdims=True)
        cand = jnp.where(cur == mv, lane, 2 ** 30)
        mi = jnp.min(cand, axis=1, keepdims=True)
        ids = jnp.where(lane == k, mi, ids)
        mask = mask & (lane != mi)
    id_ref[...] = ids


def _cmsel(CM):
    return pl.pallas_call(
        _cmsel_body,
        grid=(N // RB,),
        in_specs=[pl.BlockSpec((RB, 128), lambda i: (i, 0))],
        out_specs=pl.BlockSpec((RB, 128), lambda i: (i, 0)),
        out_shape=jax.ShapeDtypeStruct((N, 128), jnp.int32),
    )(CM)


CGB = 320


@functools.lru_cache(maxsize=1)
def _mk_sc_cgather():
  return functools.partial(
    pl.kernel,
    out_type=jax.ShapeDtypeStruct((N * K, 128), jnp.float32),
    mesh=_mesh(),
    scratch_types=[pltpu.VMEM((RPT, 128), jnp.int32),
                   pltpu.VMEM((RPT * K,), jnp.int32),
                   pltpu.VMEM((CGB, 128), jnp.float32),
                   pltpu.SemaphoreType.DMA],
  )(_sc_cgather)


def _sc_cgather(id_hbm, d2_hbm, c_hbm, idbuf, ilist, rows, sem):
    wid = _wid()
    r0 = wid * RPT
    nrows = jnp.where(wid == NW - 1, TAIL, RPT)
    lanes = lax.iota(jnp.int32, 16)

    @pl.when(wid < NW - 1)
    def _():
        pltpu.sync_copy(id_hbm.at[pl.ds(r0, RPT)], idbuf)

    @pl.when(wid == NW - 1)
    def _():
        pltpu.sync_copy(id_hbm.at[pl.ds(r0, TAIL)], idbuf.at[pl.ds(0, TAIL)])

    def mkidx(i, carry):
        idv = idbuf[i, pl.ds(0, 16)]
        ilist[pl.ds(i * K, 16)] = idv + jnp.full((16,), 80, jnp.int32) * (r0 + i)
        return carry

    lax.fori_loop(0, nrows, mkidx, jnp.int32(0))

    def gat(c, carry):
        pltpu.async_copy(d2_hbm.at[ilist.at[pl.ds(c * CGB, CGB)]], rows, sem).wait()
        pltpu.sync_copy(rows, c_hbm.at[pl.ds(r0 * K + c * CGB, CGB)])
        return carry

    lax.fori_loop(0, nrows * K // CGB, gat, jnp.int32(0))


def _ksel_body(c_ref, id_ref, nbr_ref):
    cv = c_ref[...]
    lane = lax.broadcasted_iota(jnp.int32, (RB, K, 128), 2)
    cols = []
    for j in range(K):
        cid = id_ref[:, j:j + 1]
        cols.append((cid * 128 + lane[:, 0, :])[:, None, :])
    colid = jnp.concatenate(cols, axis=1)
    mask = jnp.ones((RB, K, 128), jnp.bool_)
    out = jnp.zeros((RB, 128), jnp.int32)
    olane = lax.broadcasted_iota(jnp.int32, (RB, 128), 1)
    for k in range(K):
        cur = jnp.where(mask, cv, BIGPOS)
        mv = jnp.min(cur, axis=(1, 2))[:, None, None]
        cand = jnp.where(cur == mv, colid, 2 ** 30)
        mi = jnp.min(cand, axis=(1, 2))[:, None, None]
        out = jnp.where(olane == k, mi[:, :, 0], out)
        mask = mask & (colid != mi)
    nbr_ref[...] = out[:, 0:K]


def _ksel(C, ids):
    return pl.pallas_call(
        _ksel_body,
        grid=(N // RB,),
        in_specs=[pl.BlockSpec((RB, K, 128), lambda i: (i, 0, 0)),
                  pl.BlockSpec((RB, 128), lambda i: (i, 0))],
        out_specs=pl.BlockSpec((RB, K), lambda i: (i, 0)),
        out_shape=jax.ShapeDtypeStruct((N, K), jnp.int32),
    )(C, ids)


def kernel(x, en, theta_W, theta_b, phi_W, phi_b, theta_en_params, phi_en_params, edge_index):
    src = edge_index[0]
    dst = edge_index[1]
    tflat = []
    for w, b in theta_en_params:
        tflat += [w, b[None, :]]
    pflat = []
    for w, b in phi_en_params:
        pflat += [w, b[None, :]]

    P2, Cdst, Csrc = _prep(x, en, phi_W, phi_b[None, :], pflat)
    xd, F, p2s = _mk_sc_gather()(x, -x, Cdst, Csrc, P2, src, dst)
    exx, hh = _edge(xd, F, p2s, theta_W, theta_b[None, :], tflat)
    Z = jnp.zeros((5008, DX), jnp.float32)
    SD = _mk_sc_scatter()(hh, dst, Z)
    en_new = _finen(SD)
    M = jax.ops.segment_max(exx, dst, num_segments=N)
    x_new = jnp.where(jnp.isneginf(M), 0.0, M)
    sq = jnp.sum(x_new * x_new, axis=1)
    dist = sq[:, None] + sq[None, :] - 2.0 * (x_new @ x_new.T)
    _, nbr = jax.lax.top_k(-dist, K)
    src_new = nbr.reshape(-1)
    dst_new = jnp.repeat(jnp.arange(N, dtype=src_new.dtype), K)
    edge_index_new = jnp.stack([src_new, dst_new])
    return x_new, en_new, edge_index_new


# speedup vs baseline: 3.8651x; 3.0081x over previous
import functools

import jax
import jax.numpy as jnp
from jax import lax
from jax.experimental import pallas as pl
from jax.experimental.pallas import tpu as pltpu, tpu_sc as plsc

N = 10000
E = 320000
K = 16
DX = 128
DEN = 16

NC = 2
NS = 16
NW = NC * NS
EW = E // NW

NPAD = 10240
NCHUNK = 80
RPT = 320
TAIL = N - 31 * RPT
BIGNEG = -3.0e38
BIGPOS = 3.0e38

def _mesh():
    return plsc.VectorSubcoreMesh(core_axis_name="c", subcore_axis_name="s")


def _wid():
    return lax.axis_index("s") * NC + lax.axis_index("c")


def _prep_body(x_ref, en_ref, pw_ref, pb_ref, w0, b0, w1, b1, w2, b2, w3, b3,
               p2_ref, cdst_ref, csrc_ref):
    x = x_ref[...]
    p2_ref[...] = jnp.dot(x, pw_ref[...].T, preferred_element_type=jnp.float32) + pb_ref[...]
    en = en_ref[...]
    h = en
    for (w, b) in ((w0, b0), (w1, b1), (w2, b2)):
        h = jax.nn.relu(jnp.dot(h, w[...].T, preferred_element_type=jnp.float32) + b[...])
    p = jnp.dot(h, w3[...].T, preferred_element_type=jnp.float32) + b3[...]
    z96 = jnp.zeros((N, 96), jnp.float32)
    z112 = jnp.zeros((N, 112), jnp.float32)
    cdst_ref[...] = jnp.concatenate([en, z112], axis=1)
    csrc_ref[...] = jnp.concatenate([-en, p, z96], axis=1)


def _prep(x, en, phi_W, phi_b2, pflat):
    return pl.pallas_call(
        _prep_body,
        out_shape=(
            jax.ShapeDtypeStruct((N, DX), jnp.float32),
            jax.ShapeDtypeStruct((N, DX), jnp.float32),
            jax.ShapeDtypeStruct((N, DX), jnp.float32),
        ),
    )(x, en, phi_W, phi_b2, *pflat)


GCH = 400


@functools.lru_cache(maxsize=1)
def _mk_sc_gather():
  return functools.partial(
    pl.kernel,
    out_type=[jax.ShapeDtypeStruct((E, DX), jnp.float32),
              jax.ShapeDtypeStruct((E, DX), jnp.float32),
              jax.ShapeDtypeStruct((E, DX), jnp.float32)],
    mesh=_mesh(),
    scratch_types=[pltpu.VMEM((GCH,), jnp.int32),
                   pltpu.VMEM((GCH,), jnp.int32),
                   pltpu.VMEM((GCH, DX), jnp.float32),
                   pltpu.SemaphoreType.DMA],
  )(_sc_gather)


def _sc_gather(x_hbm, xneg_hbm, cdst_hbm, csrc_hbm, p2_hbm, src_hbm, dst_hbm,
               xd_hbm, f_hbm, p2s_hbm, sidx, didx, rows, sem):
    base0 = _wid() * EW

    def chunk(c, carry):
        base = base0 + c * GCH
        pltpu.sync_copy(src_hbm.at[pl.ds(base, GCH)], sidx)
        pltpu.sync_copy(dst_hbm.at[pl.ds(base, GCH)], didx)
        pltpu.async_copy(cdst_hbm.at[didx], rows, sem).wait()
        pltpu.async_copy(csrc_hbm.at[sidx], rows, sem, add=True).wait()
        pltpu.sync_copy(rows, f_hbm.at[pl.ds(base, GCH)])
        pltpu.async_copy(x_hbm.at[didx], rows, sem).wait()
        pltpu.async_copy(xneg_hbm.at[sidx], rows, sem, add=True).wait()
        pltpu.sync_copy(rows, xd_hbm.at[pl.ds(base, GCH)])
        pltpu.async_copy(p2_hbm.at[sidx], rows, sem).wait()
        pltpu.sync_copy(rows, p2s_hbm.at[pl.ds(base, GCH)])
        return carry

    lax.fori_loop(0, EW // GCH, chunk, jnp.int32(0))


BE = 8000


def _edge_body(xd_ref, f_ref, p2s_ref, tw_ref, tb_ref, w0, b0, w1, b1, w2, b2, w3, b3,
               exx_ref, hh_ref):
    exx_ref[...] = (jnp.dot(xd_ref[...], tw_ref[...].T,
                            preferred_element_type=jnp.float32) + tb_ref[...]) + p2s_ref[...]
    f = f_ref[...]
    h = f[:, 0:DEN]
    for (w, b) in ((w0, b0), (w1, b1), (w2, b2)):
        h = jax.nn.relu(jnp.dot(h, w[...].T, preferred_element_type=jnp.float32) + b[...])
    h = jnp.dot(h, w3[...].T, preferred_element_type=jnp.float32) + b3[...]
    h = h + f[:, DEN:2 * DEN]
    ones = jnp.ones((BE, 1), jnp.float32)
    zeros = jnp.zeros((BE, DX - DEN - 1), jnp.float32)
    hh_ref[...] = jnp.concatenate([h, ones, zeros], axis=1)


def _edge(xd, F, p2s, theta_W, theta_b2, tflat):
    nb = E // BE
    bs_e = pl.BlockSpec((BE, DX), lambda i: (i, 0))
    specs = [bs_e, bs_e, bs_e, pl.BlockSpec(theta_W.shape, lambda i: (0, 0)),
             pl.BlockSpec(theta_b2.shape, lambda i: (0, 0))]
    for t in tflat:
        specs.append(pl.BlockSpec(t.shape, lambda i: (0, 0)))
    return pl.pallas_call(
        _edge_body,
        grid=(nb,),
        in_specs=specs,
        out_specs=(bs_e, bs_e),
        out_shape=(jax.ShapeDtypeStruct((E, DX), jnp.float32),
                   jax.ShapeDtypeStruct((E, DX), jnp.float32)),
    )(xd, F, p2s, theta_W, theta_b2, *tflat)


SCH = 400
HALF = 5000
AROWS = 5008
ZR2 = 312


@functools.lru_cache(maxsize=1)
def _mk_sc_scatter():
  return functools.partial(
    pl.kernel,
    out_type=jax.ShapeDtypeStruct((NC, N, DX), jnp.float32),
    mesh=_mesh(),
    scratch_types=[pltpu.VMEM((SCH,), jnp.int32),
                   pltpu.VMEM((SCH,), jnp.int32),
                   pltpu.VMEM((SCH, DX), jnp.float32),
                   pltpu.VMEM_SHARED((AROWS, DX), jnp.float32),
                   pltpu.SemaphoreType.DMA],
  )(_sc_scatter)


def _sc_scatter(hh_hbm, dst_hbm, z_hbm, out_hbm, didx, lidx, rows, acc, sem):
    cid = lax.axis_index("c")
    sid = lax.axis_index("s")
    zbase = sid * ZR2
    base0 = _wid() * EW

    for p in range(2):
        pltpu.sync_copy(z_hbm.at[pl.ds(zbase, ZR2)], acc.at[pl.ds(zbase, ZR2)])

        @pl.when(sid == 0)
        def _():
            pltpu.sync_copy(z_hbm.at[pl.ds(NS * ZR2, AROWS - NS * ZR2)],
                            acc.at[pl.ds(NS * ZR2, AROWS - NS * ZR2)])

        plsc.subcore_barrier()
        lov = jnp.full((16,), HALF * p, jnp.int32)
        hiv = lov + HALF
        trash = jnp.full((16,), HALF, jnp.int32)

        def chunk(c, carry):
            base = base0 + c * SCH
            pltpu.sync_copy(dst_hbm.at[pl.ds(base, SCH)], didx)
            pltpu.sync_copy(hh_hbm.at[pl.ds(base, SCH)], rows)

            def remap(j, c2):
                d16 = didx[pl.ds(j * 16, 16)]
                ok = (d16 >= lov) & (d16 < hiv)
                lidx[pl.ds(j * 16, 16)] = jnp.where(ok, d16 - lov, trash)
                return c2

            lax.fori_loop(0, SCH // 16, remap, jnp.int32(0))
            pltpu.async_copy(rows, acc.at[lidx], sem, add=True).wait()
            return carry

        lax.fori_loop(0, EW // SCH, chunk, jnp.int32(0))
        plsc.subcore_barrier()
        obase = p * HALF + zbase
        pltpu.sync_copy(acc.at[pl.ds(zbase, ZR2)], out_hbm.at[cid, pl.ds(obase, ZR2)])

        @pl.when(sid == 0)
        def _():
            pltpu.sync_copy(acc.at[pl.ds(NS * ZR2, HALF - NS * ZR2)],
                            out_hbm.at[cid, pl.ds(p * HALF + NS * ZR2, HALF - NS * ZR2)])

        plsc.subcore_barrier()


def _finx_body(m_ref, xnew_ref, xpad_ref):
    m = m_ref[...]
    xn = jnp.where(m <= BIGNEG / 2, 0.0, m)
    xnew_ref[...] = xn
    xpad_ref[0:N, :] = xn
    xpad_ref[N:NPAD, :] = jnp.zeros((NPAD - N, DX), jnp.float32)


def _finx(M):
    return pl.pallas_call(
        _finx_body,
        out_shape=(jax.ShapeDtypeStruct((N, DX), jnp.float32),
                   jax.ShapeDtypeStruct((NPAD, DX), jnp.float32)),
    )(M)


def _finen_body(sd_ref, en_ref):
    s = sd_ref[0] + sd_ref[1]
    deg = s[:, DEN:DEN + 1]
    en_ref[...] = s[:, 0:DEN] / jnp.maximum(deg, 1.0)


def _finen(SD):
    return pl.pallas_call(
        _finen_body,
        out_shape=jax.ShapeDtypeStruct((N, DEN), jnp.float32),
    )(SD)


RB = 200


def _knn_body(xb_ref, xc_ref, sqb_ref, sq2_ref, nbr_ref):
    xb = xb_ref[...]
    sqb = sqb_ref[...]
    lane = lax.broadcasted_iota(jnp.int32, (RB, 128), 1)
    parts = []
    for c in range(NCHUNK):
        xc = xc_ref[pl.ds(c * 128, 128), :]
        g = jnp.dot(xb, xc.T, preferred_element_type=jnp.float32)
        sqc = sq2_ref[c:c + 1, :]
        d = (sqb + sqc) - 2.0 * g
        if c == NCHUNK - 2:
            d = jnp.where(lane < 16, d, BIGPOS)
        elif c == NCHUNK - 1:
            d = jnp.full((RB, 128), BIGPOS, jnp.float32)
        parts.append(d)
    dmat = jnp.concatenate(parts, axis=1)
    col = lax.broadcasted_iota(jnp.int32, (RB, NPAD), 1)
    olane = lax.broadcasted_iota(jnp.int32, (RB, 128), 1)
    out = jnp.zeros((RB, 128), jnp.int32)
    for k in range(K):
        mv = jnp.min(dmat, axis=1, keepdims=True)
        cand = jnp.where(dmat == mv, col, 2 ** 30)
        mi = jnp.min(cand, axis=1, keepdims=True)
        out = jnp.where(olane == k, mi, out)
        dmat = jnp.where(col == mi, BIGPOS, dmat)
    nbr_ref[...] = out[:, 0:K]


def _knn(xpad, sqcol, sq2):
    return pl.pallas_call(
        _knn_body,
        grid=(N // RB,),
        in_specs=[pl.BlockSpec((RB, DX), lambda i: (i, 0)),
                  pl.BlockSpec((NPAD, DX), lambda i: (0, 0)),
                  pl.BlockSpec((RB, 1), lambda i: (i, 0)),
                  pl.BlockSpec((NCHUNK, 128), lambda i: (0, 0))],
        out_specs=pl.BlockSpec((RB, K), lambda i: (i, 0)),
        out_shape=jax.ShapeDtypeStruct((N, K), jnp.int32),
    )(xpad, xpad, sqcol, sq2)


def _cmsel_body(cm_ref, id_ref):
    cm = cm_ref[...]
    lane = lax.broadcasted_iota(jnp.int32, (RB, 128), 1)
    mask = jnp.ones((RB, 128), jnp.bool_)
    ids = jnp.zeros((RB, 128), jnp.int32)
    for k in range(K):
        cur = jnp.where(mask, cm, BIGPOS)
        mv = jnp.min(cur, axis=1, keepdims=True)
        cand = jnp.where(cur == mv, lane, 2 ** 30)
        mi = jnp.min(cand, axis=1, keepdims=True)
        ids = jnp.where(lane == k, mi, ids)
        mask = mask & (lane != mi)
    id_ref[...] = ids


def _cmsel(CM):
    return pl.pallas_call(
        _cmsel_body,
        grid=(N // RB,),
        in_specs=[pl.BlockSpec((RB, 128), lambda i: (i, 0))],
        out_specs=pl.BlockSpec((RB, 128), lambda i: (i, 0)),
        out_shape=jax.ShapeDtypeStruct((N, 128), jnp.int32),
    )(CM)


CGB = 320


@functools.lru_cache(maxsize=1)
def _mk_sc_cgather():
  return functools.partial(
    pl.kernel,
    out_type=jax.ShapeDtypeStruct((N * K, 128), jnp.float32),
    mesh=_mesh(),
    scratch_types=[pltpu.VMEM((RPT, 128), jnp.int32),
                   pltpu.VMEM((RPT * K,), jnp.int32),
                   pltpu.VMEM((CGB, 128), jnp.float32),
                   pltpu.SemaphoreType.DMA],
  )(_sc_cgather)


def _sc_cgather(id_hbm, d2_hbm, c_hbm, idbuf, ilist, rows, sem):
    wid = _wid()
    r0 = wid * RPT
    nrows = jnp.where(wid == NW - 1, TAIL, RPT)
    lanes = lax.iota(jnp.int32, 16)

    @pl.when(wid < NW - 1)
    def _():
        pltpu.sync_copy(id_hbm.at[pl.ds(r0, RPT)], idbuf)

    @pl.when(wid == NW - 1)
    def _():
        pltpu.sync_copy(id_hbm.at[pl.ds(r0, TAIL)], idbuf.at[pl.ds(0, TAIL)])

    def mkidx(i, carry):
        idv = idbuf[i, pl.ds(0, 16)]
        ilist[pl.ds(i * K, 16)] = idv + jnp.full((16,), 80, jnp.int32) * (r0 + i)
        return carry

    lax.fori_loop(0, nrows, mkidx, jnp.int32(0))

    def gat(c, carry):
        pltpu.async_copy(d2_hbm.at[ilist.at[pl.ds(c * CGB, CGB)]], rows, sem).wait()
        pltpu.sync_copy(rows, c_hbm.at[pl.ds(r0 * K + c * CGB, CGB)])
        return carry

    lax.fori_loop(0, nrows * K // CGB, gat, jnp.int32(0))


def _ksel_body(c_ref, id_ref, nbr_ref):
    cv = c_ref[...]
    lane = lax.broadcasted_iota(jnp.int32, (RB, K, 128), 2)
    cols = []
    for j in range(K):
        cid = id_ref[:, j:j + 1]
        cols.append((cid * 128 + lane[:, 0, :])[:, None, :])
    colid = jnp.concatenate(cols, axis=1)
    mask = jnp.ones((RB, K, 128), jnp.bool_)
    out = jnp.zeros((RB, 128), jnp.int32)
    olane = lax.broadcasted_iota(jnp.int32, (RB, 128), 1)
    for k in range(K):
        cur = jnp.where(mask, cv, BIGPOS)
        mv = jnp.min(cur, axis=(1, 2))[:, None, None]
        cand = jnp.where(cur == mv, colid, 2 ** 30)
        mi = jnp.min(cand, axis=(1, 2))[:, None, None]
        out = jnp.where(olane == k, mi[:, :, 0], out)
        mask = mask & (colid != mi)
    nbr_ref[...] = out[:, 0:K]


def _ksel(C, ids):
    return pl.pallas_call(
        _ksel_body,
        grid=(N // RB,),
        in_specs=[pl.BlockSpec((RB, K, 128), lambda i: (i, 0, 0)),
                  pl.BlockSpec((RB, 128), lambda i: (i, 0))],
        out_specs=pl.BlockSpec((RB, K), lambda i: (i, 0)),
        out_shape=jax.ShapeDtypeStruct((N, K), jnp.int32),
    )(C, ids)


def kernel(x, en, theta_W, theta_b, phi_W, phi_b, theta_en_params, phi_en_params, edge_index):
    src = edge_index[0]
    dst = edge_index[1]
    tflat = []
    for w, b in theta_en_params:
        tflat += [w, b[None, :]]
    pflat = []
    for w, b in phi_en_params:
        pflat += [w, b[None, :]]

    P2, Cdst, Csrc = _prep(x, en, phi_W, phi_b[None, :], pflat)
    xd, F, p2s = _mk_sc_gather()(x, -x, Cdst, Csrc, P2, src, dst)
    exx, hh = _edge(xd, F, p2s, theta_W, theta_b[None, :], tflat)
    Z = jnp.zeros((5008, DX), jnp.float32)
    SD = _mk_sc_scatter()(hh, dst, Z)
    en_new = _finen(SD)
    M = jax.ops.segment_max(exx, dst, num_segments=N)
    x_new, xpad = _finx(M)
    sq = jnp.sum(x_new * x_new, axis=1)
    sqpad = jnp.concatenate([sq, jnp.zeros((NPAD - N,), jnp.float32)])
    nbr = _knn(xpad, sqpad.reshape(NPAD, 1), sqpad.reshape(NCHUNK, 128))
    src_new = nbr.reshape(-1)
    dst_new = jnp.repeat(jnp.arange(N, dtype=src_new.dtype), K)
    edge_index_new = jnp.stack([src_new, dst_new])
    return x_new, en_new, edge_index_new
